# baseline (speedup 1.0000x reference)
import functools

import jax
import jax.numpy as jnp
from jax import lax
from jax.experimental import pallas as pl
from jax.experimental.pallas import tpu as pltpu
from jax.experimental.pallas import tpu_sc as plsc

B = 16384
D = 64
DI = 32
NW = 32
ROWS_PER_W = B // NW
CHUNK = 128
CHUNKS_PER_W = ROWS_PER_W // CHUNK
IDX_ROWS = B // CHUNK
V = 100000
DOT_W = 100096
NSTEP = 8


def _tc_proj_body(uT, iT, wdanO, w2O, o_p, o_d, dacc):
  pid = pl.program_id(0)
  d0 = (((0,), (0,)), ((), ()))

  @pl.when(pid == 0)
  def _init():
    o_p[...] = jnp.zeros_like(o_p)
    dacc[...] = jnp.zeros_like(dacc)

  part = lax.dot_general(wdanO[0], uT[...], d0,
                         preferred_element_type=jnp.float32)
  o_p[:, pl.ds(0, V)] += part
  dacc[...] += iT[...] * w2O[0]

  @pl.when(pid == NSTEP - 1)
  def _fin():
    o_d[0:1, pl.ds(0, V)] = jnp.sum(dacc[...], axis=0, keepdims=True)


def _tc_proj(uT, iT, wdanO, w2O):
  return pl.pallas_call(
      _tc_proj_body,
      grid=(NSTEP,),
      in_specs=[pl.BlockSpec((8, V), lambda i: (i, 0))] * 2
      + [pl.BlockSpec((1, 8, DI), lambda i: (i, 0, 0)),
         pl.BlockSpec((1, 8, 1), lambda i: (i, 0, 0))],
      out_specs=[pl.BlockSpec((DI, DOT_W), lambda i: (0, 0)),
                 pl.BlockSpec((8, DOT_W), lambda i: (0, 0))],
      out_shape=[jax.ShapeDtypeStruct((DI, DOT_W), jnp.float32),
                 jax.ShapeDtypeStruct((8, DOT_W), jnp.float32)],
      scratch_shapes=[pltpu.VMEM((8, V), jnp.float32)],
  )(uT, iT, wdanO, w2O)


def _sc_gather_body(projS, projT, dotS, dotT, i_su, i_tu, i_sp, i_sn,
                    i_tp, i_tn, o_u, o_d, iu_v, it_v, val_v, sem):
  wid = lax.axis_index("s") * 2 + lax.axis_index("c")
  base = pl.multiple_of(wid * CHUNKS_PER_W, CHUNKS_PER_W)
  row0 = pl.multiple_of(wid * ROWS_PER_W, ROWS_PER_W)

  def fire_drain_store(src_row, idx_v, dst):
    copies = [
        pltpu.async_copy(src_row.at[idx_v.at[j]],
                         val_v.at[pl.ds(j * CHUNK, CHUNK)], sem)
        for j in range(CHUNKS_PER_W)
    ]
    for c in copies:
      c.wait()
    pltpu.sync_copy(val_v, dst)

  for proj, idx, f0 in ((projS, i_su, 0), (projT, i_tu, DI)):
    pltpu.sync_copy(idx.at[pl.ds(base, CHUNKS_PER_W), :], iu_v)
    for f in range(DI):
      fire_drain_store(proj.at[f], iu_v,
                       o_u.at[f0 + f, pl.ds(row0, ROWS_PER_W)])
  for r, (dvec, idx) in enumerate(((dotS, i_sp), (dotS, i_sn),
                                   (dotT, i_tp), (dotT, i_tn))):
    pltpu.sync_copy(idx.at[pl.ds(base, CHUNKS_PER_W), :], it_v)
    fire_drain_store(dvec.at[0], it_v, o_d.at[r, pl.ds(row0, ROWS_PER_W)])


def _sc_gather(projS, projT, dotS, dotT, i_su, i_tu, i_sp, i_sn, i_tp, i_tn):
  mesh = plsc.VectorSubcoreMesh(core_axis_name="c", subcore_axis_name="s")
  k = functools.partial(
      pl.kernel,
      mesh=mesh,
      compiler_params=pltpu.CompilerParams(use_tc_tiling_on_sc=False),
      out_type=[jax.ShapeDtypeStruct((2 * DI, B), jnp.float32),
                jax.ShapeDtypeStruct((8, B), jnp.float32)],
      scratch_types=[
          pltpu.VMEM((CHUNKS_PER_W, CHUNK), jnp.int32),
          pltpu.VMEM((CHUNKS_PER_W, CHUNK), jnp.int32),
          pltpu.VMEM((ROWS_PER_W,), jnp.float32),
          pltpu.SemaphoreType.DMA,
      ],
  )(_sc_gather_body)
  return k(projS, projT, dotS, dotT, i_su, i_tu, i_sp, i_sn, i_tp, i_tn)


BLK = 2048


def _tc_body(uproj, dots, bdanC, wencT, bencC, wdecT, bdecC,
             w_user, bvec, out):
  dn = (((1,), (0,)), ((), ()))
  dot = functools.partial(lax.dot_general,
                          preferred_element_type=jnp.float32)
  up = uproj[...]
  sdiT = jnp.maximum(up[0:DI] + bdanC[...], 0.0)
  tdiT = jnp.maximum(up[DI:2 * DI] + bdanC[...], 0.0)
  scdT = dot(wdecT[...],
             jnp.maximum(dot(wencT[...], sdiT, dn) + bencC[...], 0.0),
             dn) + bdecC[...]
  tcdT = dot(wdecT[...],
             jnp.maximum(dot(wencT[...], tdiT, dn) + bencC[...], 0.0),
             dn) + bdecC[...]
  ustack = jnp.concatenate([sdiT, scdT, tdiT, tcdT], axis=0)
  uh = dot(w_user[...], ustack, dn)
  a_s, c_s, a_t, c_t = uh[0:1], uh[1:2], uh[2:3], uh[3:4]
  d = dots[...]
  p_s, n_s, p_t, n_t = d[0:1], d[1:2], d[2:3], d[3:4]
  bs = bvec[0, 0]
  bt = bvec[0, 1]
  out[...] = jnp.concatenate(
      [a_s + p_s + bs, a_t + p_t + bt,
       a_s + n_s + bs, a_t + n_t + bt,
       c_s + p_s + bs, c_s + n_s + bs,
       c_t + p_t + bt, c_t + n_t + bt], axis=0)


def _tc_heads(uproj, dots, bdanC, wencT, bencC, wdecT, bdecC, w_user, bvec):
  full = lambda a: pl.BlockSpec(a.shape, lambda i: (0,) * a.ndim)
  return pl.pallas_call(
      _tc_body,
      grid=(B // BLK,),
      in_specs=[pl.BlockSpec((2 * DI, BLK), lambda i: (0, i)),
                pl.BlockSpec((8, BLK), lambda i: (0, i))]
      + [full(w) for w in (bdanC, wencT, bencC, wdecT, bdecC,
                           w_user, bvec)],
      out_specs=pl.BlockSpec((8, BLK), lambda i: (0, i)),
      out_shape=jax.ShapeDtypeStruct((8, B), jnp.float32),
  )(uproj, dots, bdanC, wencT, bencC, wdecT, bdecC, w_user, bvec)


def kernel(SInterBatch, TInterBatch, SUserTable, TUserTable, SItemTable,
           TItemTable, W_dan, b_dan, W_srec, b_srec, W_trec, b_trec,
           W_enc, b_enc, W_dec, b_dec):
  idx = lambda a: a.astype(jnp.int32).reshape(IDX_ROWS, CHUNK)

  wdanO = W_dan.reshape(8, 8, DI)
  projS, dotS = _tc_proj(SUserTable.T, SItemTable.T, wdanO,
                         W_srec[32:].reshape(8, 8, 1))
  projT, dotT = _tc_proj(TUserTable.T, TItemTable.T, wdanO,
                         W_trec[32:].reshape(8, 8, 1))
  uproj, dots = _sc_gather(
      projS, projT, dotS, dotT,
      idx(SInterBatch[0]), idx(TInterBatch[0]),
      idx(SInterBatch[1]), idx(SInterBatch[2]),
      idx(TInterBatch[1]), idx(TInterBatch[2]))

  z32 = jnp.zeros((1, 32), jnp.float32)
  ws1T, wt1T = W_srec[:32].T, W_trec[:32].T
  w_user = jnp.concatenate([
      jnp.concatenate([ws1T, z32, z32, z32], axis=1),
      jnp.concatenate([z32, ws1T, z32, z32], axis=1),
      jnp.concatenate([z32, z32, wt1T, z32], axis=1),
      jnp.concatenate([z32, z32, z32, wt1T], axis=1)], axis=0)
  bvec = jnp.stack([b_srec[0], b_trec[0]]).reshape(1, 2)

  outs = _tc_heads(
      uproj, dots,
      b_dan.reshape(-1, 1), W_enc.T, b_enc.reshape(-1, 1),
      W_dec.T, b_dec.reshape(-1, 1), w_user, bvec)
  return outs.reshape(8 * B, 1)


# speedup vs baseline: 1.2829x; 1.2829x over previous
import functools

import jax
import jax.numpy as jnp
from jax import lax
from jax.experimental import pallas as pl
from jax.experimental.pallas import tpu as pltpu
from jax.experimental.pallas import tpu_sc as plsc

B = 16384
D = 64
NW = 32
ROWS_PER_W = B // NW
CHUNK = 128
CHUNKS_PER_W = ROWS_PER_W // CHUNK
IDX_ROWS = B // CHUNK
V = 100000
DOT_W = 100096
NSTEP = 8


def _tc_dots_body(siT, tiT, ws2C, wt2C, out, accS, accT):
  pid = pl.program_id(0)

  @pl.when(pid == 0)
  def _init():
    accS[...] = jnp.zeros_like(accS)
    accT[...] = jnp.zeros_like(accT)

  accS[...] += siT[...] * ws2C[pl.ds(pid * 8, 8), :]
  accT[...] += tiT[...] * wt2C[pl.ds(pid * 8, 8), :]

  @pl.when(pid == NSTEP - 1)
  def _fin():
    out[0:1, pl.ds(0, V)] = jnp.sum(accS[...], axis=0, keepdims=True)
    out[1:2, pl.ds(0, V)] = jnp.sum(accT[...], axis=0, keepdims=True)


def _tc_dots(siT, tiT, ws2C, wt2C):
  full = lambda a: pl.BlockSpec(a.shape, lambda i: (0,) * a.ndim)
  return pl.pallas_call(
      _tc_dots_body,
      grid=(NSTEP,),
      in_specs=[pl.BlockSpec((8, V), lambda i: (i, 0))] * 2
      + [full(ws2C), full(wt2C)],
      out_specs=pl.BlockSpec((8, DOT_W), lambda i: (0, 0)),
      out_shape=jax.ShapeDtypeStruct((8, DOT_W), jnp.float32),
      scratch_shapes=[pltpu.VMEM((8, V), jnp.float32)] * 2,
  )(siT, tiT, ws2C, wt2C)


def _sc_mesh_kernel(body, out_types, scratch_types):
  mesh = plsc.VectorSubcoreMesh(core_axis_name="c", subcore_axis_name="s")
  return functools.partial(
      pl.kernel,
      mesh=mesh,
      compiler_params=pltpu.CompilerParams(use_tc_tiling_on_sc=False),
      out_type=out_types,
      scratch_types=scratch_types + [pltpu.SemaphoreType.DMA],
  )(body)


def _sc_dots_body(dots, i_sp, i_sn, i_tp, i_tn, o_d, idx_v, val_v, sem):
  wid = lax.axis_index("s") * 2 + lax.axis_index("c")
  base = wid * CHUNKS_PER_W
  row0 = wid * ROWS_PER_W
  for r, (src_row, idx) in enumerate(((0, i_sp), (0, i_sn),
                                      (1, i_tp), (1, i_tn))):
    pltpu.sync_copy(idx.at[pl.ds(base, CHUNKS_PER_W), :], idx_v)
    copies = [
        pltpu.async_copy(dots.at[src_row].at[idx_v.at[j]],
                         val_v.at[pl.ds(j * CHUNK, CHUNK)], sem)
        for j in range(CHUNKS_PER_W)
    ]
    for c in copies:
      c.wait()
    pltpu.sync_copy(val_v, o_d.at[r, pl.ds(row0, ROWS_PER_W)])


def _sc_dots(dots8, i_sp, i_sn, i_tp, i_tn):
  k = _sc_mesh_kernel(
      _sc_dots_body,
      [jax.ShapeDtypeStruct((8, B), jnp.float32)],
      [pltpu.VMEM((CHUNKS_PER_W, CHUNK), jnp.int32),
       pltpu.VMEM((ROWS_PER_W,), jnp.float32)])
  return k(dots8, i_sp, i_sn, i_tp, i_tn)


def _user_gather_body(tab, idx, out, col, idx_v, rows_v, sem):
  wid = lax.axis_index("s") * 2 + lax.axis_index("c")
  base = wid * CHUNKS_PER_W
  row0 = wid * ROWS_PER_W
  pltpu.sync_copy(idx.at[pl.ds(base, CHUNKS_PER_W), :], idx_v)
  copies = [
      pltpu.async_copy(tab.at[idx_v.at[j]],
                       rows_v.at[pl.ds(j * CHUNK, CHUNK), :], sem)
      for j in range(CHUNKS_PER_W)
  ]
  for c in copies:
    c.wait()
  pltpu.sync_copy(rows_v, out.at[pl.ds(row0, ROWS_PER_W), pl.ds(col, D)])


def _sc_su_body(su_t, i_su, o_u, idx_v, rows_v, sem):
  _user_gather_body(su_t, i_su, o_u, 0, idx_v, rows_v, sem)


def _sc_tu_body(tu_t, i_tu, o_u, idx_v, rows_v, sem):
  _user_gather_body(tu_t, i_tu, o_u, D, idx_v, rows_v, sem)


def _sc_user(body, tab, idx):
  k = _sc_mesh_kernel(
      body,
      [jax.ShapeDtypeStruct((B, 2 * D), jnp.float32)],
      [pltpu.VMEM((CHUNKS_PER_W, CHUNK), jnp.int32),
       pltpu.VMEM((ROWS_PER_W, D), jnp.float32)])
  return k(tab, idx)


BLK = 2048


def _tc_body(us, ut, dots, wdanT, bdanC, wencT, bencC, wdecT, bdecC,
             w_user, bvec, out):
  dn = (((1,), (0,)), ((), ()))
  dt = (((1,), (1,)), ((), ()))
  dot = functools.partial(lax.dot_general,
                          preferred_element_type=jnp.float32)
  su = us[:, :D]
  tu = ut[:, D:]
  sdiT = jnp.maximum(dot(wdanT[...], su, dt) + bdanC[...], 0.0)
  tdiT = jnp.maximum(dot(wdanT[...], tu, dt) + bdanC[...], 0.0)
  scdT = dot(wdecT[...],
             jnp.maximum(dot(wencT[...], sdiT, dn) + bencC[...], 0.0),
             dn) + bdecC[...]
  tcdT = dot(wdecT[...],
             jnp.maximum(dot(wencT[...], tdiT, dn) + bencC[...], 0.0),
             dn) + bdecC[...]
  ustack = jnp.concatenate([sdiT, scdT, tdiT, tcdT], axis=0)
  uh = dot(w_user[...], ustack, dn)
  a_s, c_s, a_t, c_t = uh[0:1], uh[1:2], uh[2:3], uh[3:4]
  d = dots[...]
  p_s, n_s, p_t, n_t = d[0:1], d[1:2], d[2:3], d[3:4]
  bs = bvec[0, 0]
  bt = bvec[0, 1]
  out[...] = jnp.concatenate(
      [a_s + p_s + bs, a_t + p_t + bt,
       a_s + n_s + bs, a_t + n_t + bt,
       c_s + p_s + bs, c_s + n_s + bs,
       c_t + p_t + bt, c_t + n_t + bt], axis=0)


def _tc_heads(us, ut, dots, wdanT, bdanC, wencT, bencC, wdecT, bdecC,
              w_user, bvec):
  row_spec = pl.BlockSpec((BLK, 2 * D), lambda i: (i, 0))
  full = lambda a: pl.BlockSpec(a.shape, lambda i: (0,) * a.ndim)
  return pl.pallas_call(
      _tc_body,
      grid=(B // BLK,),
      in_specs=[row_spec] * 2 + [pl.BlockSpec((8, BLK), lambda i: (0, i))]
      + [full(w) for w in (wdanT, bdanC, wencT, bencC, wdecT, bdecC,
                           w_user, bvec)],
      out_specs=pl.BlockSpec((8, BLK), lambda i: (0, i)),
      out_shape=jax.ShapeDtypeStruct((8, B), jnp.float32),
  )(us, ut, dots, wdanT, bdanC, wencT, bencC, wdecT, bdecC, w_user, bvec)


def kernel(SInterBatch, TInterBatch, SUserTable, TUserTable, SItemTable,
           TItemTable, W_dan, b_dan, W_srec, b_srec, W_trec, b_trec,
           W_enc, b_enc, W_dec, b_dec):
  idx = lambda a: a.astype(jnp.int32).reshape(IDX_ROWS, CHUNK)

  dots8 = _tc_dots(SItemTable.T, TItemTable.T, W_srec[32:], W_trec[32:])
  dots = _sc_dots(dots8,
                  idx(SInterBatch[1]), idx(SInterBatch[2]),
                  idx(TInterBatch[1]), idx(TInterBatch[2]))
  (us,) = _sc_user(_sc_su_body, SUserTable, idx(SInterBatch[0]))
  (ut,) = _sc_user(_sc_tu_body, TUserTable, idx(TInterBatch[0]))

  z32 = jnp.zeros((1, 32), jnp.float32)
  ws1T, wt1T = W_srec[:32].T, W_trec[:32].T
  w_user = jnp.concatenate([
      jnp.concatenate([ws1T, z32, z32, z32], axis=1),
      jnp.concatenate([z32, ws1T, z32, z32], axis=1),
      jnp.concatenate([z32, z32, wt1T, z32], axis=1),
      jnp.concatenate([z32, z32, z32, wt1T], axis=1)], axis=0)
  bvec = jnp.stack([b_srec[0], b_trec[0]]).reshape(1, 2)

  outs = _tc_heads(
      us, ut, dots[0],
      W_dan.T, b_dan.reshape(-1, 1), W_enc.T, b_enc.reshape(-1, 1),
      W_dec.T, b_dec.reshape(-1, 1), w_user, bvec)
  return outs.reshape(8 * B, 1)
